# baseline (speedup 1.0000x reference)
import functools

import jax
import jax.numpy as jnp
from jax import lax
from jax.experimental import pallas as pl
from jax.experimental.pallas import tpu as pltpu
from jax.experimental.pallas import tpu_sc as plsc

N = 10000
D = 256
H = D // 2
E = 160000
NT = 16
EPT = E // NT
C = 80
RPT = E // NT // C
NB = 5
NPT = 640
SUB = 80
BN = 1024
NP = 10240


def _sc_body(xh, eidx, acc_out, deg_out, sidx, sidx2, didx, dld, bufs, ones_v,
             zbuf, zdeg, accum, deg, gsems, ssems, semd, semz):
    c = lax.axis_index("c")
    s = lax.axis_index("s")

    z32 = jnp.zeros((32,), jnp.bfloat16)
    z16 = jnp.zeros((16,), jnp.float32)
    o16 = jnp.ones((16,), jnp.float32)

    def zrow(i, _):
        for j in range(H // 32):
            zbuf[i, pl.ds(j * 32, 32)] = z32
        return _
    lax.fori_loop(0, 16, zrow, 0)
    for g in range(112 // 16):
        ones_v[pl.ds(g * 16, 16)] = o16
    for g in range(SUB // 16):
        zdeg[pl.ds(g * 16, 16)] = z16

    for k in range(NPT // SUB):
        nb = s * NPT + k * SUB

        @pl.when(nb < N)
        def _():
            for m in range(SUB // 16):
                pltpu.async_copy(zbuf, accum.at[pl.ds(nb + m * 16, 16), :],
                                 semz)
            pltpu.async_copy(zdeg, deg.at[pl.ds(nb, SUB)], semz)
    for k in range(NPT // SUB):
        nb = s * NPT + k * SUB

        @pl.when(nb < N)
        def _():
            for m in range(SUB // 16):
                pltpu.make_async_copy(
                    zbuf, accum.at[pl.ds(nb + m * 16, 16), :], semz).wait()
            pltpu.make_async_copy(zdeg, deg.at[pl.ds(nb, SUB)], semz).wait()

    plsc.subcore_barrier()

    pltpu.sync_copy(eidx.at[0, pl.ds(s * EPT, EPT)], sidx)
    pltpu.sync_copy(eidx.at[1, pl.ds(s * EPT, EPT)], dld)

    def sbody(g, carry):
        for k in range(C // 16):
            sl = pl.ds(g * C + k * 16, 16)
            sidx2[g, pl.ds(k * 16, 16)] = sidx[sl] * 2 + c
            didx[g, pl.ds(k * 16, 16)] = dld[sl]
        return carry
    lax.fori_loop(0, RPT, sbody, 0)

    half = RPT // 2

    def deg_scatter(j):
        mine = lax.select(c == 0, j < half, j >= half)

        @pl.when(mine)
        def _():
            pltpu.async_copy(ones_v.at[pl.ds(0, C)], deg.at[didx.at[j]],
                             semd, add=True)

    for r in range(2):
        pltpu.async_copy(xh.at[sidx2.at[r]], bufs.at[r], gsems[r])

    def chunk(i, carry):
        for r in range(NB):
            j = NB * i + r
            b2 = (r + 2) % NB

            @pl.when(j >= 3)
            def _():
                pltpu.make_async_copy(bufs.at[b2], accum.at[didx.at[0]],
                                      ssems[b2]).wait()

            @pl.when(j + 2 < RPT)
            def _():
                pltpu.async_copy(xh.at[sidx2.at[j + 2]], bufs.at[b2],
                                 gsems[b2])
            pltpu.make_async_copy(xh.at[sidx2.at[j]],
                                  bufs.at[r], gsems[r]).wait()
            pltpu.async_copy(bufs.at[r], accum.at[didx.at[j]], ssems[r],
                             add=True)
            deg_scatter(j)
        return carry
    lax.fori_loop(0, RPT // NB, chunk, 0)
    for j in (RPT - 3, RPT - 2, RPT - 1):
        pltpu.make_async_copy(bufs.at[j % NB], accum.at[didx.at[0]],
                              ssems[j % NB]).wait()

    ndeg = lax.select(c == 0, half, RPT - half)

    def deg_drain(i, carry):
        @pl.when(i < ndeg)
        def _():
            pltpu.make_async_copy(ones_v.at[pl.ds(0, C)], deg.at[didx.at[0]],
                                  semd).wait()
        return carry
    lax.fori_loop(0, RPT - half, deg_drain, 0)

    plsc.subcore_barrier()

    for k in range(NPT // SUB):
        nb = s * NPT + k * SUB

        @pl.when(nb < N)
        def _():
            pltpu.sync_copy(accum.at[pl.ds(nb, SUB), :],
                            acc_out.at[c, pl.ds(nb, SUB), :])
            pltpu.sync_copy(deg.at[pl.ds(nb, SUB)],
                            deg_out.at[c, pl.ds(nb, SUB)])


@jax.jit
def _sc_aggregate(xh, eidx):
    mesh = plsc.VectorSubcoreMesh(core_axis_name="c", subcore_axis_name="s")
    f = functools.partial(
        pl.kernel,
        mesh=mesh,
        compiler_params=pltpu.CompilerParams(use_tc_tiling_on_sc=False),
        out_type=(
            jax.ShapeDtypeStruct((2, N, H), jnp.bfloat16),
            jax.ShapeDtypeStruct((2, NP), jnp.float32),
        ),
        scratch_types=[
            pltpu.VMEM((EPT,), jnp.int32),
            pltpu.VMEM((RPT, C), jnp.int32),
            pltpu.VMEM((RPT, C), jnp.int32),
            pltpu.VMEM((EPT,), jnp.int32),
            pltpu.VMEM((NB, C, H), jnp.bfloat16),
            pltpu.VMEM((112,), jnp.float32),
            pltpu.VMEM((16, H), jnp.bfloat16),
            pltpu.VMEM((SUB,), jnp.float32),
            pltpu.VMEM_SHARED((N, H), jnp.bfloat16),
            pltpu.VMEM_SHARED((N,), jnp.float32),
            [pltpu.SemaphoreType.DMA] * NB,
            [pltpu.SemaphoreType.DMA] * NB,
            pltpu.SemaphoreType.DMA,
            pltpu.SemaphoreType.DMA,
        ],
    )(_sc_body)
    return f(xh, eidx)


def _tc_body(x_ref, a_ref, deg_ref, out_ref):
    i = pl.program_id(0)
    x = x_ref[...]
    acc = jnp.concatenate([a_ref[0], a_ref[1]], axis=1).astype(jnp.float32)
    off = pl.multiple_of(i * BN, 128)
    d0 = deg_ref[0, pl.ds(off, BN)]
    d1 = deg_ref[1, pl.ds(off, BN)]
    inv = 1.0 / (d0 + d1 + 1.0)
    red = (acc + x) * inv[:, None]
    out_ref[:, :D] = x
    out_ref[:, D:] = red


@jax.jit
def _tc_epilogue(x, acc, deg):
    return pl.pallas_call(
        _tc_body,
        grid=(NP // BN,),
        in_specs=[
            pl.BlockSpec((BN, D), lambda i: (i, 0)),
            pl.BlockSpec((2, BN, H), lambda i: (0, i, 0)),
            pl.BlockSpec((2, NP), lambda i: (0, 0)),
        ],
        out_specs=pl.BlockSpec((BN, 2 * D), lambda i: (i, 0)),
        out_shape=jax.ShapeDtypeStruct((N, 2 * D), jnp.float32),
    )(x, acc, deg)


def kernel(x, edge_index):
    xh = x.astype(jnp.bfloat16).reshape(2 * N, H)
    acc, deg = _sc_aggregate(xh, edge_index)
    return _tc_epilogue(x, acc, deg)


# speedup vs baseline: 1.0458x; 1.0458x over previous
import functools

import jax
import jax.numpy as jnp
from jax import lax
from jax.experimental import pallas as pl
from jax.experimental.pallas import tpu as pltpu
from jax.experimental.pallas import tpu_sc as plsc

N = 10000
D = 256
H = D // 2
E = 160000
NT = 16
EPT = E // NT
C = 80
RPT = E // NT // C
NB = 5
NPT = 640
SUB = 80
BN = 1024
NP = 10240


def _sc_body(xh, eidx, acc_out, deg_out, sidx, sidx2, didx, dld, bufs, ones_v,
             zbuf, zdeg, accum, deg, gsems, semd, semz):
    c = lax.axis_index("c")
    s = lax.axis_index("s")

    z32 = jnp.zeros((32,), jnp.bfloat16)
    z16 = jnp.zeros((16,), jnp.float32)
    o16 = jnp.ones((16,), jnp.float32)

    def zrow(i, _):
        for j in range(H // 32):
            zbuf[i, pl.ds(j * 32, 32)] = z32
        return _
    lax.fori_loop(0, 16, zrow, 0)
    for g in range(112 // 16):
        ones_v[pl.ds(g * 16, 16)] = o16
    for g in range(SUB // 16):
        zdeg[pl.ds(g * 16, 16)] = z16

    for k in range(NPT // SUB):
        nb = s * NPT + k * SUB

        @pl.when(nb < N)
        def _():
            for m in range(SUB // 16):
                pltpu.async_copy(zbuf, accum.at[pl.ds(nb + m * 16, 16), :],
                                 semz)
            pltpu.async_copy(zdeg, deg.at[pl.ds(nb, SUB)], semz)
    for k in range(NPT // SUB):
        nb = s * NPT + k * SUB

        @pl.when(nb < N)
        def _():
            for m in range(SUB // 16):
                pltpu.make_async_copy(
                    zbuf, accum.at[pl.ds(nb + m * 16, 16), :], semz).wait()
            pltpu.make_async_copy(zdeg, deg.at[pl.ds(nb, SUB)], semz).wait()

    plsc.subcore_barrier()

    pltpu.sync_copy(eidx.at[0, pl.ds(s * EPT, EPT)], sidx)
    pltpu.sync_copy(eidx.at[1, pl.ds(s * EPT, EPT)], dld)

    def sbody(g, carry):
        for k in range(C // 16):
            sl = pl.ds(g * C + k * 16, 16)
            sidx2[g, pl.ds(k * 16, 16)] = sidx[sl] * 2 + c
            didx[g, pl.ds(k * 16, 16)] = dld[sl]
        return carry
    lax.fori_loop(0, RPT, sbody, 0)

    half = RPT // 2

    def deg_scatter(j):
        mine = lax.select(c == 0, j < half, j >= half)

        @pl.when(mine)
        def _():
            pltpu.async_copy(ones_v.at[pl.ds(0, C)], deg.at[didx.at[j]],
                             semd, add=True)

    for r in range(NB):
        pltpu.async_copy(xh.at[sidx2.at[r]], bufs.at[r], gsems[r])

    def chunk(i, carry):
        for r in range(NB):
            j = NB * i + r
            pltpu.make_async_copy(xh.at[sidx2.at[j]],
                                  bufs.at[r], gsems[r]).wait()
            pltpu.sync_copy(bufs.at[r], accum.at[didx.at[j]], add=True)
            deg_scatter(j)

            @pl.when(j + NB < RPT)
            def _():
                pltpu.async_copy(xh.at[sidx2.at[j + NB]], bufs.at[r],
                                 gsems[r])
        return carry
    lax.fori_loop(0, RPT // NB, chunk, 0)

    ndeg = lax.select(c == 0, half, RPT - half)

    def deg_drain(i, carry):
        @pl.when(i < ndeg)
        def _():
            pltpu.make_async_copy(ones_v.at[pl.ds(0, C)], deg.at[didx.at[0]],
                                  semd).wait()
        return carry
    lax.fori_loop(0, RPT - half, deg_drain, 0)

    plsc.subcore_barrier()

    for k in range(NPT // SUB):
        nb = s * NPT + k * SUB

        @pl.when(nb < N)
        def _():
            pltpu.sync_copy(accum.at[pl.ds(nb, SUB), :],
                            acc_out.at[c, pl.ds(nb, SUB), :])
            pltpu.sync_copy(deg.at[pl.ds(nb, SUB)],
                            deg_out.at[c, pl.ds(nb, SUB)])


@jax.jit
def _sc_aggregate(xh, eidx):
    mesh = plsc.VectorSubcoreMesh(core_axis_name="c", subcore_axis_name="s")
    f = functools.partial(
        pl.kernel,
        mesh=mesh,
        compiler_params=pltpu.CompilerParams(use_tc_tiling_on_sc=False),
        out_type=(
            jax.ShapeDtypeStruct((2, N, H), jnp.bfloat16),
            jax.ShapeDtypeStruct((2, NP), jnp.float32),
        ),
        scratch_types=[
            pltpu.VMEM((EPT,), jnp.int32),
            pltpu.VMEM((RPT, C), jnp.int32),
            pltpu.VMEM((RPT, C), jnp.int32),
            pltpu.VMEM((EPT,), jnp.int32),
            pltpu.VMEM((NB, C, H), jnp.bfloat16),
            pltpu.VMEM((112,), jnp.float32),
            pltpu.VMEM((16, H), jnp.bfloat16),
            pltpu.VMEM((SUB,), jnp.float32),
            pltpu.VMEM_SHARED((N, H), jnp.bfloat16),
            pltpu.VMEM_SHARED((N,), jnp.float32),
            [pltpu.SemaphoreType.DMA] * NB,
            pltpu.SemaphoreType.DMA,
            pltpu.SemaphoreType.DMA,
        ],
    )(_sc_body)
    return f(xh, eidx)


def _tc_body(x_ref, a_ref, deg_ref, out_ref):
    i = pl.program_id(0)
    x = x_ref[...]
    acc = jnp.concatenate([a_ref[0], a_ref[1]], axis=1).astype(jnp.float32)
    off = pl.multiple_of(i * BN, 128)
    d0 = deg_ref[0, pl.ds(off, BN)]
    d1 = deg_ref[1, pl.ds(off, BN)]
    inv = 1.0 / (d0 + d1 + 1.0)
    red = (acc + x) * inv[:, None]
    out_ref[:, :D] = x
    out_ref[:, D:] = red


@jax.jit
def _tc_epilogue(x, acc, deg):
    return pl.pallas_call(
        _tc_body,
        grid=(NP // BN,),
        in_specs=[
            pl.BlockSpec((BN, D), lambda i: (i, 0)),
            pl.BlockSpec((2, BN, H), lambda i: (0, i, 0)),
            pl.BlockSpec((2, NP), lambda i: (0, 0)),
        ],
        out_specs=pl.BlockSpec((BN, 2 * D), lambda i: (i, 0)),
        out_shape=jax.ShapeDtypeStruct((N, 2 * D), jnp.float32),
    )(x, acc, deg)


def kernel(x, edge_index):
    xh = x.astype(jnp.bfloat16).reshape(2 * N, H)
    acc, deg = _sc_aggregate(xh, edge_index)
    return _tc_epilogue(x, acc, deg)
